# baseline (speedup 1.0000x reference)
import functools

import jax
import jax.numpy as jnp
from jax import lax
from jax.experimental import pallas as pl
from jax.experimental.pallas import tpu as pltpu
from jax.experimental.pallas import tpu_sc as plsc

DIM_ = 64
ROWS_ = 800
N_WORDS_ = 80
LANES_ = 16
SPW_ = 0.1 / (DIM_ - 1)
SCALE_ = 0.5 / N_WORDS_


def _body(table, out, idx_v, rows_v, tv_v, sem):
    lanes = lax.iota(jnp.int32, 16)
    for t in range(5):
        g = t * 16 + lanes
        d = jnp.where(g < 32, g >> 2,
                      jnp.where(g < 64, (g - 32) >> 2, (g - 64) >> 1))
        off = jnp.where(g < 32, g & 3,
                        jnp.where(g < 64, 10 + ((g - 32) & 3),
                                  20 + ((g - 64) & 1)))
        idx_v[16 * t:16 * (t + 1)] = d * 100 + off
    gather = pltpu.async_copy(table.at[idx_v], rows_v, sem)

    fzero = jnp.zeros((16,), jnp.float32)
    fone = jnp.ones((16,), jnp.float32)
    gather.wait()

    def dim_block(d, carry):
        f, acc0, acc1, acc2, acc3 = carry
        onehot = jnp.where(lanes == d, fone, fzero)
        fp_sum = fzero
        for j in range(4):
            r = 4 * d + j
            s0 = rows_v[r, 0:16]
            a0 = jnp.abs(s0)
            fp_sum = fp_sum + (jnp.where(s0 <= 0.0, a0 + 0.1, -0.1 * s0)
                               - SPW_ * a0)
            acc0 = acc0 + a0
            acc1 = acc1 + jnp.abs(rows_v[r, 16:32])
            acc2 = acc2 + jnp.abs(rows_v[r, 32:48])
            acc3 = acc3 + jnp.abs(rows_v[r, 48:64])
        fn_sum = fzero
        for j in range(4):
            r = 32 + 4 * d + j
            s0 = rows_v[r, 0:16]
            a0 = jnp.abs(s0)
            fn_sum = fn_sum + (jnp.where(s0 >= 0.0, a0 + 0.1, 0.1 * s0)
                               - SPW_ * a0)
            acc0 = acc0 + a0
            acc1 = acc1 + jnp.abs(rows_v[r, 16:32])
            acc2 = acc2 + jnp.abs(rows_v[r, 32:48])
            acc3 = acc3 + jnp.abs(rows_v[r, 48:64])
        fu_sum = (jnp.abs(rows_v[64 + 2 * d, 0:16])
                  + jnp.abs(rows_v[65 + 2 * d, 0:16]))
        f = f + (fp_sum + fn_sum + 2.0 * fu_sum) * onehot
        return f, acc0, acc1, acc2, acc3

    f, acc0, acc1, acc2, acc3 = lax.fori_loop(
        0, 8, dim_block, (fzero, fzero, fzero, fzero, fzero))

    total_vec = f + SPW_ * ((acc0 + acc1) + (acc2 + acc3))
    total = jnp.float32(0.0)
    for j in range(16):
        total = total + total_vec[j]
    total = total * SCALE_
    tv_v[...] = jnp.full((16,), total, jnp.float32)
    pltpu.sync_copy(tv_v.at[0:1], out)


_sc_call = functools.partial(
    pl.kernel,
    mesh=plsc.VectorSubcoreMesh(core_axis_name="c", subcore_axis_name="s",
                                num_cores=1, num_subcores=1),
    out_type=jax.ShapeDtypeStruct((1,), jnp.float32),
    compiler_params=pltpu.CompilerParams(use_tc_tiling_on_sc=False,
                                         allow_input_fusion=[True]),
    scratch_types=[
        pltpu.VMEM((N_WORDS_,), jnp.int32),
        pltpu.VMEM((N_WORDS_, DIM_), jnp.float32),
        pltpu.VMEM((LANES_,), jnp.float32),
        pltpu.SemaphoreType.DMA,
    ],
)(_body)


@jax.jit
def kernel(embeddings):
    out = _sc_call(embeddings[:ROWS_])
    return jnp.reshape(out, ())


# speedup vs baseline: 1.0027x; 1.0027x over previous
import functools

import jax
import jax.numpy as jnp
from jax import lax
from jax.experimental import pallas as pl
from jax.experimental.pallas import tpu as pltpu
from jax.experimental.pallas import tpu_sc as plsc

DIM_ = 64
ROWS_ = 728
N_WORDS_ = 80
LANES_ = 16
SPW_ = 0.1 / (DIM_ - 1)
SCALE_ = 0.5 / N_WORDS_


def _body(table, out, idx_v, rows_v, tv_v, sem):
    lanes = lax.iota(jnp.int32, 16)
    for t in range(5):
        g = t * 16 + lanes
        d = jnp.where(g < 32, g >> 2,
                      jnp.where(g < 64, (g - 32) >> 2, (g - 64) >> 1))
        off = jnp.where(g < 32, g & 3,
                        jnp.where(g < 64, 10 + ((g - 32) & 3),
                                  20 + ((g - 64) & 1)))
        idx_v[16 * t:16 * (t + 1)] = d * 100 + off
    gather = pltpu.async_copy(table.at[idx_v], rows_v, sem)

    fzero = jnp.zeros((16,), jnp.float32)
    fone = jnp.ones((16,), jnp.float32)
    gather.wait()

    def dim_block(d, carry):
        f, acc0, acc1, acc2, acc3 = carry
        onehot = jnp.where(lanes == d, fone, fzero)
        fp_sum = fzero
        for j in range(4):
            r = 4 * d + j
            s0 = rows_v[r, 0:16]
            a0 = jnp.abs(s0)
            fp_sum = fp_sum + (jnp.where(s0 <= 0.0, a0 + 0.1, -0.1 * s0)
                               - SPW_ * a0)
            acc0 = acc0 + a0
            acc1 = acc1 + jnp.abs(rows_v[r, 16:32])
            acc2 = acc2 + jnp.abs(rows_v[r, 32:48])
            acc3 = acc3 + jnp.abs(rows_v[r, 48:64])
        fn_sum = fzero
        for j in range(4):
            r = 32 + 4 * d + j
            s0 = rows_v[r, 0:16]
            a0 = jnp.abs(s0)
            fn_sum = fn_sum + (jnp.where(s0 >= 0.0, a0 + 0.1, 0.1 * s0)
                               - SPW_ * a0)
            acc0 = acc0 + a0
            acc1 = acc1 + jnp.abs(rows_v[r, 16:32])
            acc2 = acc2 + jnp.abs(rows_v[r, 32:48])
            acc3 = acc3 + jnp.abs(rows_v[r, 48:64])
        fu_sum = (jnp.abs(rows_v[64 + 2 * d, 0:16])
                  + jnp.abs(rows_v[65 + 2 * d, 0:16]))
        f = f + (fp_sum + fn_sum + 2.0 * fu_sum) * onehot
        return f, acc0, acc1, acc2, acc3

    f, acc0, acc1, acc2, acc3 = lax.fori_loop(
        0, 8, dim_block, (fzero, fzero, fzero, fzero, fzero))

    total_vec = f + SPW_ * ((acc0 + acc1) + (acc2 + acc3))
    total = jnp.float32(0.0)
    for j in range(16):
        total = total + total_vec[j]
    total = total * SCALE_
    tv_v[...] = jnp.full((16,), total, jnp.float32)
    pltpu.sync_copy(tv_v.at[0:1], out)


_sc_call = functools.partial(
    pl.kernel,
    mesh=plsc.VectorSubcoreMesh(core_axis_name="c", subcore_axis_name="s",
                                num_cores=1, num_subcores=1),
    out_type=jax.ShapeDtypeStruct((1,), jnp.float32),
    compiler_params=pltpu.CompilerParams(use_tc_tiling_on_sc=False),
    scratch_types=[
        pltpu.VMEM((N_WORDS_,), jnp.int32),
        pltpu.VMEM((N_WORDS_, DIM_), jnp.float32),
        pltpu.VMEM((LANES_,), jnp.float32),
        pltpu.SemaphoreType.DMA,
    ],
)(_body)


@jax.jit
def kernel(embeddings):
    out = _sc_call(embeddings[:ROWS_])
    return jnp.reshape(out, ())
